# baseline (speedup 1.0000x reference)
import functools

import jax
import jax.numpy as jnp
from jax import lax
from jax.experimental import pallas as pl
from jax.experimental.pallas import tpu as pltpu
from jax.experimental.pallas import tpu_sc as plsc

NUM_SEGMENTS = 16
TOTAL_TOKENS = 32768
DIM = 1024
NEG = -1e30

K_TC = 0
BLOCK_T = 4096
NW = 32
CHUNK = 32

S = NUM_SEGMENTS
NJ = DIM // 16


_GDN = lax.GatherDimensionNumbers(offset_dims=(), collapsed_slice_dims=(0,),
                                  start_index_map=(0,))


def _lane_permute(x, idx):
    return lax.gather(x, idx[:, None], dimension_numbers=_GDN,
                      slice_sizes=(1,),
                      mode=lax.GatherScatterMode.PROMISE_IN_BOUNDS)


def _lane_all_sum(x, iota16):
    for sh in (1, 2, 4, 8):
        x = x + _lane_permute(x, jnp.bitwise_xor(iota16, sh))
    return x


def _sc_body(seq_ref, att_ref, ids_ref, m_out, d_out, acc_out,
             xb0, xb1, att_v, idv, lb, accb, mb, db, sem0, sem1):
    tw = idv.shape[0]
    nc = tw // CHUNK
    wid = lax.axis_index("s") * 2 + lax.axis_index("c")
    base = wid * tw

    pltpu.sync_copy(att_ref, att_v)
    pltpu.sync_copy(ids_ref.at[pl.ds(base, tw)], idv)

    zeros16 = jnp.zeros((16,), jnp.float32)
    mb[...] = jnp.full((16,), NEG, jnp.float32)
    db[...] = zeros16

    def zbody(k, _):
        accb[pl.ds(k * 16, 16)] = zeros16
        return 0
    lax.fori_loop(0, S * NJ, zbody, 0)

    iota16 = lax.iota(jnp.int32, 16)

    def process(xbuf, ch):
        off = ch * CHUNK

        def tok_logit(i, bmv):
            va = [zeros16, zeros16, zeros16, zeros16]
            for j in range(NJ):
                va[j % 4] = va[j % 4] + (xbuf[i, pl.ds(j * 16, 16)]
                                         * att_v[pl.ds(j * 16, 16)])
            lvec = plsc.cumsum((va[0] + va[1]) + (va[2] + va[3]))
            lb[i] = lvec[15]
            idvec = plsc.load_gather(idv, [jnp.broadcast_to(off + i, (16,))])
            lv = jnp.where(iota16 == idvec, lvec[15], NEG)
            return jnp.maximum(bmv, lv)

        bmv = lax.fori_loop(0, CHUNK, tok_logit,
                            jnp.full((16,), NEG, jnp.float32))

        m_old = mb[...]
        m_new = jnp.maximum(m_old, bmv)
        cvec = jnp.exp(m_old - m_new)
        mb[...] = m_new
        db[...] = db[...] * cvec

        @pl.when(jnp.any(cvec < 1.0))
        def _rescale():
            def row(s, _):
                cs = plsc.cumsum(jnp.where(iota16 == s, cvec, 0.0))
                @pl.when(cs[15] < 1.0)
                def _():
                    def rj(j, _2):
                        sl = pl.ds(s * DIM + j * 16, 16)
                        accb[sl] = accb[sl] * cs[15]
                        return 0
                    lax.fori_loop(0, NJ, rj, 0)
                return 0
            lax.fori_loop(0, S, row, 0)

        mv = mb[...]

        def tok_acc(i, _):
            idvec = plsc.load_gather(idv, [jnp.broadcast_to(off + i, (16,))])
            l = lb[i]
            m_s = plsc.load_gather(mb, [idvec])
            w_vec = jnp.exp(l - m_s)
            oh = iota16 == idvec
            db[...] = db[...] + jnp.where(oh, w_vec, 0.0)
            rb = idvec[0] * DIM
            for j in range(NJ):
                sl = pl.ds(rb + j * 16, 16)
                accb[sl] = accb[sl] + w_vec * xbuf[i, pl.ds(j * 16, 16)]
            return 0

        lax.fori_loop(0, CHUNK, tok_acc, 0)

    def pair(p, _):
        t0 = base + (2 * p) * CHUNK
        cp0 = pltpu.async_copy(seq_ref.at[pl.ds(t0, CHUNK)], xb0, sem0)
        cp1 = pltpu.async_copy(seq_ref.at[pl.ds(t0 + CHUNK, CHUNK)], xb1, sem1)
        cp0.wait()
        process(xb0, 2 * p)
        cp1.wait()
        process(xb1, 2 * p + 1)
        return 0

    lax.fori_loop(0, nc // 2, pair, 0)

    pltpu.sync_copy(mb, m_out.at[wid])
    pltpu.sync_copy(db, d_out.at[wid])
    pltpu.sync_copy(accb, acc_out.at[wid])


def _make_sc_call(tokens_sc):
    tw = tokens_sc // NW
    mesh = plsc.VectorSubcoreMesh(core_axis_name="c", subcore_axis_name="s")
    return pl.kernel(
        _sc_body,
        out_type=(
            jax.ShapeDtypeStruct((NW, 16), jnp.float32),
            jax.ShapeDtypeStruct((NW, 16), jnp.float32),
            jax.ShapeDtypeStruct((NW, S * DIM), jnp.float32),
        ),
        mesh=mesh,
        scratch_types=[
            pltpu.VMEM((CHUNK, DIM), jnp.float32),
            pltpu.VMEM((CHUNK, DIM), jnp.float32),
            pltpu.VMEM((DIM,), jnp.float32),
            pltpu.VMEM((tw,), jnp.int32),
            pltpu.SMEM((CHUNK,), jnp.float32),
            pltpu.VMEM((S * DIM,), jnp.float32),
            pltpu.VMEM((16,), jnp.float32),
            pltpu.VMEM((16,), jnp.float32),
            pltpu.SemaphoreType.DMA,
            pltpu.SemaphoreType.DMA,
        ],
        compiler_params=pltpu.CompilerParams(needs_layout_passes=False),
    )


def _tc_body(x_ref, att_ref, idr_ref, m_out, d_out, acc_out,
             m_ref, d_ref, acc_ref):
    i = pl.program_id(0)
    nb = pl.num_programs(0)
    T = BLOCK_T

    @pl.when(i == 0)
    def _init():
        m_ref[...] = jnp.full((S, 1), NEG, jnp.float32)
        d_ref[...] = jnp.zeros((S, 1), jnp.float32)
        acc_ref[...] = jnp.zeros((S, DIM), jnp.float32)

    x = x_ref[...]
    a = att_ref[...]
    idr = idr_ref[0]

    l = lax.dot_general(a, x, (((1,), (1,)), ((), ())),
                        preferred_element_type=jnp.float32)
    seg_st = lax.broadcasted_iota(jnp.int32, (S, T), 0)
    mask = seg_st == idr
    lm = jnp.where(mask, l, NEG)
    bm = jnp.max(lm, axis=1, keepdims=True)
    m_old = m_ref[...]
    m_new = jnp.maximum(m_old, bm)
    c = jnp.exp(m_old - m_new)
    pw = jnp.exp(jnp.where(mask, l - m_new, NEG))
    d_ref[...] = d_ref[...] * c + jnp.sum(pw, axis=1, keepdims=True)
    m_ref[...] = m_new
    acc_ref[...] = (acc_ref[...] * c
                    + jnp.dot(pw, x, preferred_element_type=jnp.float32))

    @pl.when(i == nb - 1)
    def _fin():
        m_out[...] = m_ref[...]
        d_out[...] = d_ref[...]
        acc_out[...] = acc_ref[...]


def _tc_partial(seq_tc, att_row, idr):
    nb = seq_tc.shape[0] // BLOCK_T
    return pl.pallas_call(
        _tc_body,
        grid=(nb,),
        in_specs=[
            pl.BlockSpec((BLOCK_T, DIM), lambda i: (i, 0)),
            pl.BlockSpec((1, DIM), lambda i: (0, 0)),
            pl.BlockSpec((1, 1, BLOCK_T), lambda i: (i, 0, 0)),
        ],
        out_specs=[
            pl.BlockSpec((S, 1), lambda i: (0, 0)),
            pl.BlockSpec((S, 1), lambda i: (0, 0)),
            pl.BlockSpec((S, DIM), lambda i: (0, 0)),
        ],
        out_shape=[
            jax.ShapeDtypeStruct((S, 1), jnp.float32),
            jax.ShapeDtypeStruct((S, 1), jnp.float32),
            jax.ShapeDtypeStruct((S, DIM), jnp.float32),
        ],
        scratch_shapes=[
            pltpu.VMEM((S, 1), jnp.float32),
            pltpu.VMEM((S, 1), jnp.float32),
            pltpu.VMEM((S, DIM), jnp.float32),
        ],
        compiler_params=pltpu.CompilerParams(
            dimension_semantics=("arbitrary",)),
    )(seq_tc, att_row, idr)


def _merge_body(msc_ref, dsc_ref, accsc_ref, mtc_ref, dtc_ref, acctc_ref,
                out_ref):
    eye = (lax.broadcasted_iota(jnp.int32, (S, S), 0)
           == lax.broadcasted_iota(jnp.int32, (S, S), 1))
    ones_row = jnp.ones((1, S), jnp.float32)
    mtc_row = jnp.dot(ones_row, jnp.where(eye, mtc_ref[...], 0.0),
                      preferred_element_type=jnp.float32)
    dtc_row = jnp.dot(ones_row, jnp.where(eye, dtc_ref[...], 0.0),
                      preferred_element_type=jnp.float32)
    m_sc = msc_ref[...]
    M = jnp.maximum(jnp.max(m_sc, axis=0, keepdims=True), mtc_row)
    sc_sc = jnp.exp(m_sc - M)
    sc_tc = jnp.exp(mtc_row - M)
    D = (jnp.sum(dsc_ref[...] * sc_sc, axis=0, keepdims=True)
         + dtc_row * sc_tc)

    acc0 = jnp.dot(jnp.where(eye, sc_tc, 0.0), acctc_ref[...],
                   preferred_element_type=jnp.float32)

    def w_body(w, acc):
        mw = msc_ref[pl.ds(w, 1), :]
        scw = jnp.exp(mw - M)
        return acc + jnp.dot(jnp.where(eye, scw, 0.0),
                             accsc_ref[pl.ds(w * 16, 16), :],
                             preferred_element_type=jnp.float32)

    acc = lax.fori_loop(0, NW, w_body, acc0)
    dinv = jnp.where(eye, jnp.where(D > 0, 1.0 / D, 0.0), 0.0)
    out_ref[...] = jnp.dot(dinv, acc, preferred_element_type=jnp.float32)


def _merge(m_sc, d_sc, acc_sc, m_tc, d_tc, acc_tc):
    return pl.pallas_call(
        _merge_body,
        out_shape=jax.ShapeDtypeStruct((S, DIM), jnp.float32),
    )(m_sc, d_sc, acc_sc.reshape(NW * S, DIM), m_tc, d_tc, acc_tc)


@jax.jit
def kernel(seq, att, segment_ids):
    ids = segment_ids.astype(jnp.int32)
    att_row = att.reshape(1, DIM)
    att_flat = att.reshape(DIM)

    if K_TC > 0:
        idr = ids[:K_TC].reshape(K_TC // BLOCK_T, 1, BLOCK_T)
        m_tc, d_tc, acc_tc = _tc_partial(seq[:K_TC], att_row, idr)
    else:
        m_tc = jnp.full((S, 1), NEG, jnp.float32)
        d_tc = jnp.zeros((S, 1), jnp.float32)
        acc_tc = jnp.zeros((S, DIM), jnp.float32)

    tokens_sc = TOTAL_TOKENS - K_TC
    if tokens_sc > 0:
        m_sc, d_sc, acc_sc = _make_sc_call(tokens_sc)(
            seq[K_TC:], att_flat, ids[K_TC:])
    else:
        m_sc = jnp.full((NW, 16), NEG, jnp.float32)
        d_sc = jnp.zeros((NW, 16), jnp.float32)
        acc_sc = jnp.zeros((NW, S * DIM), jnp.float32)

    return _merge(m_sc, d_sc, acc_sc, m_tc, d_tc, acc_tc)


# speedup vs baseline: 1.1379x; 1.1379x over previous
import functools

import jax
import jax.numpy as jnp
from jax import lax
from jax.experimental import pallas as pl
from jax.experimental.pallas import tpu as pltpu
from jax.experimental.pallas import tpu_sc as plsc

NUM_SEGMENTS = 16
TOTAL_TOKENS = 32768
DIM = 1024
NEG = -1e30

K_TC = 0
BLOCK_T = 4096
NW = 32
CHUNK = 32

S = NUM_SEGMENTS
NJ = DIM // 16


_GDN = lax.GatherDimensionNumbers(offset_dims=(), collapsed_slice_dims=(0,),
                                  start_index_map=(0,))


def _lane_permute(x, idx):
    return lax.gather(x, idx[:, None], dimension_numbers=_GDN,
                      slice_sizes=(1,),
                      mode=lax.GatherScatterMode.PROMISE_IN_BOUNDS)


def _lane_all_sum(x, iota16):
    for sh in (1, 2, 4, 8):
        x = x + _lane_permute(x, jnp.bitwise_xor(iota16, sh))
    return x


def _sc_body(seq_ref, att_ref, ids_ref, m_out, d_out, acc_out,
             xb0, xb1, att_v, idv, lb, accb, mb, db, sem0, sem1):
    tw = idv.shape[0]
    nc = tw // CHUNK
    wid = lax.axis_index("s") * 2 + lax.axis_index("c")
    base = wid * tw

    pltpu.sync_copy(att_ref, att_v)
    pltpu.sync_copy(ids_ref.at[pl.ds(base, tw)], idv)

    zeros16 = jnp.zeros((16,), jnp.float32)
    mb[...] = jnp.full((16,), NEG, jnp.float32)
    db[...] = zeros16

    def zbody(k, _):
        accb[pl.ds(k * 16, 16)] = zeros16
        return 0
    lax.fori_loop(0, S * NJ, zbody, 0)

    iota16 = lax.iota(jnp.int32, 16)

    def process(xbuf, ch):
        off = ch * CHUNK

        def tok_logit(i, bmv):
            va = [zeros16, zeros16, zeros16, zeros16]
            for j in range(NJ):
                va[j % 4] = va[j % 4] + (xbuf[i, pl.ds(j * 16, 16)]
                                         * att_v[pl.ds(j * 16, 16)])
            lvec = plsc.cumsum((va[0] + va[1]) + (va[2] + va[3]))
            lb[i] = lvec[15]
            idvec = plsc.load_gather(idv, [jnp.broadcast_to(off + i, (16,))])
            lv = jnp.where(iota16 == idvec, lvec[15], NEG)
            return jnp.maximum(bmv, lv)

        bmv = lax.fori_loop(0, CHUNK, tok_logit,
                            jnp.full((16,), NEG, jnp.float32))

        m_old = mb[...]
        m_new = jnp.maximum(m_old, bmv)
        cvec = jnp.exp(m_old - m_new)
        mb[...] = m_new
        db[...] = db[...] * cvec

        @pl.when(jnp.any(cvec < 1.0))
        def _rescale():
            def row(s, _):
                cs = plsc.cumsum(jnp.where(iota16 == s, cvec, 0.0))
                @pl.when(cs[15] < 1.0)
                def _():
                    def rj(j, _2):
                        sl = pl.ds(s * DIM + j * 16, 16)
                        accb[sl] = accb[sl] * cs[15]
                        return 0
                    lax.fori_loop(0, NJ, rj, 0)
                return 0
            lax.fori_loop(0, S, row, 0)

        mv = mb[...]

        def tok_acc(i, _):
            idvec = plsc.load_gather(idv, [jnp.broadcast_to(off + i, (16,))])
            l = lb[i]
            m_s = plsc.load_gather(mb, [idvec])
            w_vec = jnp.exp(l - m_s)
            oh = iota16 == idvec
            db[...] = db[...] + jnp.where(oh, w_vec, 0.0)
            for j in range(NJ):
                idx = idvec * DIM + (j * 16) + iota16
                plsc.addupdate_scatter(
                    accb, [idx], w_vec * xbuf[i, pl.ds(j * 16, 16)])
            return 0

        lax.fori_loop(0, CHUNK, tok_acc, 0)

    def pair(p, _):
        t0 = base + (2 * p) * CHUNK
        cp0 = pltpu.async_copy(seq_ref.at[pl.ds(t0, CHUNK)], xb0, sem0)
        cp1 = pltpu.async_copy(seq_ref.at[pl.ds(t0 + CHUNK, CHUNK)], xb1, sem1)
        cp0.wait()
        process(xb0, 2 * p)
        cp1.wait()
        process(xb1, 2 * p + 1)
        return 0

    lax.fori_loop(0, nc // 2, pair, 0)

    pltpu.sync_copy(mb, m_out.at[wid])
    pltpu.sync_copy(db, d_out.at[wid])
    pltpu.sync_copy(accb, acc_out.at[wid])


def _make_sc_call(tokens_sc):
    tw = tokens_sc // NW
    mesh = plsc.VectorSubcoreMesh(core_axis_name="c", subcore_axis_name="s")
    return pl.kernel(
        _sc_body,
        out_type=(
            jax.ShapeDtypeStruct((NW, 16), jnp.float32),
            jax.ShapeDtypeStruct((NW, 16), jnp.float32),
            jax.ShapeDtypeStruct((NW, S * DIM), jnp.float32),
        ),
        mesh=mesh,
        scratch_types=[
            pltpu.VMEM((CHUNK, DIM), jnp.float32),
            pltpu.VMEM((CHUNK, DIM), jnp.float32),
            pltpu.VMEM((DIM,), jnp.float32),
            pltpu.VMEM((tw,), jnp.int32),
            pltpu.SMEM((CHUNK,), jnp.float32),
            pltpu.VMEM((S * DIM,), jnp.float32),
            pltpu.VMEM((16,), jnp.float32),
            pltpu.VMEM((16,), jnp.float32),
            pltpu.SemaphoreType.DMA,
            pltpu.SemaphoreType.DMA,
        ],
        compiler_params=pltpu.CompilerParams(needs_layout_passes=False),
    )


def _tc_body(x_ref, att_ref, idr_ref, m_out, d_out, acc_out,
             m_ref, d_ref, acc_ref):
    i = pl.program_id(0)
    nb = pl.num_programs(0)
    T = BLOCK_T

    @pl.when(i == 0)
    def _init():
        m_ref[...] = jnp.full((S, 1), NEG, jnp.float32)
        d_ref[...] = jnp.zeros((S, 1), jnp.float32)
        acc_ref[...] = jnp.zeros((S, DIM), jnp.float32)

    x = x_ref[...]
    a = att_ref[...]
    idr = idr_ref[0]

    l = lax.dot_general(a, x, (((1,), (1,)), ((), ())),
                        preferred_element_type=jnp.float32)
    seg_st = lax.broadcasted_iota(jnp.int32, (S, T), 0)
    mask = seg_st == idr
    lm = jnp.where(mask, l, NEG)
    bm = jnp.max(lm, axis=1, keepdims=True)
    m_old = m_ref[...]
    m_new = jnp.maximum(m_old, bm)
    c = jnp.exp(m_old - m_new)
    pw = jnp.exp(jnp.where(mask, l - m_new, NEG))
    d_ref[...] = d_ref[...] * c + jnp.sum(pw, axis=1, keepdims=True)
    m_ref[...] = m_new
    acc_ref[...] = (acc_ref[...] * c
                    + jnp.dot(pw, x, preferred_element_type=jnp.float32))

    @pl.when(i == nb - 1)
    def _fin():
        m_out[...] = m_ref[...]
        d_out[...] = d_ref[...]
        acc_out[...] = acc_ref[...]


def _tc_partial(seq_tc, att_row, idr):
    nb = seq_tc.shape[0] // BLOCK_T
    return pl.pallas_call(
        _tc_body,
        grid=(nb,),
        in_specs=[
            pl.BlockSpec((BLOCK_T, DIM), lambda i: (i, 0)),
            pl.BlockSpec((1, DIM), lambda i: (0, 0)),
            pl.BlockSpec((1, 1, BLOCK_T), lambda i: (i, 0, 0)),
        ],
        out_specs=[
            pl.BlockSpec((S, 1), lambda i: (0, 0)),
            pl.BlockSpec((S, 1), lambda i: (0, 0)),
            pl.BlockSpec((S, DIM), lambda i: (0, 0)),
        ],
        out_shape=[
            jax.ShapeDtypeStruct((S, 1), jnp.float32),
            jax.ShapeDtypeStruct((S, 1), jnp.float32),
            jax.ShapeDtypeStruct((S, DIM), jnp.float32),
        ],
        scratch_shapes=[
            pltpu.VMEM((S, 1), jnp.float32),
            pltpu.VMEM((S, 1), jnp.float32),
            pltpu.VMEM((S, DIM), jnp.float32),
        ],
        compiler_params=pltpu.CompilerParams(
            dimension_semantics=("arbitrary",)),
    )(seq_tc, att_row, idr)


def _merge_body(msc_ref, dsc_ref, accsc_ref, mtc_ref, dtc_ref, acctc_ref,
                out_ref):
    eye = (lax.broadcasted_iota(jnp.int32, (S, S), 0)
           == lax.broadcasted_iota(jnp.int32, (S, S), 1))
    ones_row = jnp.ones((1, S), jnp.float32)
    mtc_row = jnp.dot(ones_row, jnp.where(eye, mtc_ref[...], 0.0),
                      preferred_element_type=jnp.float32)
    dtc_row = jnp.dot(ones_row, jnp.where(eye, dtc_ref[...], 0.0),
                      preferred_element_type=jnp.float32)
    m_sc = msc_ref[...]
    M = jnp.maximum(jnp.max(m_sc, axis=0, keepdims=True), mtc_row)
    sc_sc = jnp.exp(m_sc - M)
    sc_tc = jnp.exp(mtc_row - M)
    D = (jnp.sum(dsc_ref[...] * sc_sc, axis=0, keepdims=True)
         + dtc_row * sc_tc)

    acc0 = jnp.dot(jnp.where(eye, sc_tc, 0.0), acctc_ref[...],
                   preferred_element_type=jnp.float32)

    def w_body(w, acc):
        mw = msc_ref[pl.ds(w, 1), :]
        scw = jnp.exp(mw - M)
        return acc + jnp.dot(jnp.where(eye, scw, 0.0),
                             accsc_ref[pl.ds(w * 16, 16), :],
                             preferred_element_type=jnp.float32)

    acc = lax.fori_loop(0, NW, w_body, acc0)
    dinv = jnp.where(eye, jnp.where(D > 0, 1.0 / D, 0.0), 0.0)
    out_ref[...] = jnp.dot(dinv, acc, preferred_element_type=jnp.float32)


def _merge(m_sc, d_sc, acc_sc, m_tc, d_tc, acc_tc):
    return pl.pallas_call(
        _merge_body,
        out_shape=jax.ShapeDtypeStruct((S, DIM), jnp.float32),
    )(m_sc, d_sc, acc_sc.reshape(NW * S, DIM), m_tc, d_tc, acc_tc)


@jax.jit
def kernel(seq, att, segment_ids):
    ids = segment_ids.astype(jnp.int32)
    att_row = att.reshape(1, DIM)
    att_flat = att.reshape(DIM)

    if K_TC > 0:
        idr = ids[:K_TC].reshape(K_TC // BLOCK_T, 1, BLOCK_T)
        m_tc, d_tc, acc_tc = _tc_partial(seq[:K_TC], att_row, idr)
    else:
        m_tc = jnp.full((S, 1), NEG, jnp.float32)
        d_tc = jnp.zeros((S, 1), jnp.float32)
        acc_tc = jnp.zeros((S, DIM), jnp.float32)

    tokens_sc = TOTAL_TOKENS - K_TC
    if tokens_sc > 0:
        m_sc, d_sc, acc_sc = _make_sc_call(tokens_sc)(
            seq[K_TC:], att_flat, ids[K_TC:])
    else:
        m_sc = jnp.full((NW, 16), NEG, jnp.float32)
        d_sc = jnp.zeros((NW, 16), jnp.float32)
        acc_sc = jnp.zeros((NW, S * DIM), jnp.float32)

    return _merge(m_sc, d_sc, acc_sc, m_tc, d_tc, acc_tc)


# speedup vs baseline: 3.5228x; 3.0959x over previous
import functools

import jax
import jax.numpy as jnp
from jax import lax
from jax.experimental import pallas as pl
from jax.experimental.pallas import tpu as pltpu
from jax.experimental.pallas import tpu_sc as plsc

NUM_SEGMENTS = 16
TOTAL_TOKENS = 32768
DIM = 1024
NEG = -1e30

K_TC = 30720
BLOCK_T = 2048
NW = 32
CHUNK = 32

S = NUM_SEGMENTS
NJ = DIM // 16


_GDN = lax.GatherDimensionNumbers(offset_dims=(), collapsed_slice_dims=(0,),
                                  start_index_map=(0,))


def _lane_permute(x, idx):
    return lax.gather(x, idx[:, None], dimension_numbers=_GDN,
                      slice_sizes=(1,),
                      mode=lax.GatherScatterMode.PROMISE_IN_BOUNDS)


def _lane_all_sum(x, iota16):
    for sh in (1, 2, 4, 8):
        x = x + _lane_permute(x, jnp.bitwise_xor(iota16, sh))
    return x


def _sc_body(seq_ref, att_ref, ids_ref, m_out, d_out, acc_out,
             xb0, xb1, att_v, idv, lb, accb, mb, db, sem0, sem1):
    tw = idv.shape[0]
    nc = tw // CHUNK
    wid = lax.axis_index("s") * 2 + lax.axis_index("c")
    base = wid * tw

    pltpu.sync_copy(att_ref, att_v)
    pltpu.sync_copy(ids_ref.at[pl.ds(base, tw)], idv)

    zeros16 = jnp.zeros((16,), jnp.float32)
    mb[...] = jnp.full((16,), NEG, jnp.float32)
    db[...] = zeros16

    def zbody(k, _):
        accb[pl.ds(k * 16, 16)] = zeros16
        return 0
    lax.fori_loop(0, S * NJ, zbody, 0)

    iota16 = lax.iota(jnp.int32, 16)

    def process(xbuf, ch):
        off = ch * CHUNK

        def tok_logit(i, bmv):
            va = [zeros16, zeros16, zeros16, zeros16]
            for j in range(NJ):
                va[j % 4] = va[j % 4] + (xbuf[i, pl.ds(j * 16, 16)]
                                         * att_v[pl.ds(j * 16, 16)])
            lvec = plsc.cumsum((va[0] + va[1]) + (va[2] + va[3]))
            lb[i] = lvec[15]
            idvec = plsc.load_gather(idv, [jnp.broadcast_to(off + i, (16,))])
            lv = jnp.where(iota16 == idvec, lvec[15], NEG)
            return jnp.maximum(bmv, lv)

        bmv = lax.fori_loop(0, CHUNK, tok_logit,
                            jnp.full((16,), NEG, jnp.float32))

        m_old = mb[...]
        m_new = jnp.maximum(m_old, bmv)
        cvec = jnp.exp(m_old - m_new)
        mb[...] = m_new
        db[...] = db[...] * cvec

        @pl.when(jnp.any(cvec < 1.0))
        def _rescale():
            def row(s, _):
                cs = plsc.cumsum(jnp.where(iota16 == s, cvec, 0.0))
                @pl.when(cs[15] < 1.0)
                def _():
                    def rj(j, _2):
                        sl = pl.ds(s * DIM + j * 16, 16)
                        accb[sl] = accb[sl] * cs[15]
                        return 0
                    lax.fori_loop(0, NJ, rj, 0)
                return 0
            lax.fori_loop(0, S, row, 0)

        mv = mb[...]

        def tok_acc(i, _):
            idvec = plsc.load_gather(idv, [jnp.broadcast_to(off + i, (16,))])
            l = lb[i]
            m_s = plsc.load_gather(mb, [idvec])
            w_vec = jnp.exp(l - m_s)
            oh = iota16 == idvec
            db[...] = db[...] + jnp.where(oh, w_vec, 0.0)
            for j in range(NJ):
                idx = idvec * DIM + (j * 16) + iota16
                plsc.addupdate_scatter(
                    accb, [idx], w_vec * xbuf[i, pl.ds(j * 16, 16)])
            return 0

        lax.fori_loop(0, CHUNK, tok_acc, 0)

    def pair(p, _):
        t0 = base + (2 * p) * CHUNK
        cp0 = pltpu.async_copy(seq_ref.at[pl.ds(t0, CHUNK)], xb0, sem0)
        cp1 = pltpu.async_copy(seq_ref.at[pl.ds(t0 + CHUNK, CHUNK)], xb1, sem1)
        cp0.wait()
        process(xb0, 2 * p)
        cp1.wait()
        process(xb1, 2 * p + 1)
        return 0

    lax.fori_loop(0, nc // 2, pair, 0)

    pltpu.sync_copy(mb, m_out.at[wid])
    pltpu.sync_copy(db, d_out.at[wid])
    pltpu.sync_copy(accb, acc_out.at[wid])


def _make_sc_call(tokens_sc):
    tw = tokens_sc // NW
    mesh = plsc.VectorSubcoreMesh(core_axis_name="c", subcore_axis_name="s")
    return pl.kernel(
        _sc_body,
        out_type=(
            jax.ShapeDtypeStruct((NW, 16), jnp.float32),
            jax.ShapeDtypeStruct((NW, 16), jnp.float32),
            jax.ShapeDtypeStruct((NW, S * DIM), jnp.float32),
        ),
        mesh=mesh,
        scratch_types=[
            pltpu.VMEM((CHUNK, DIM), jnp.float32),
            pltpu.VMEM((CHUNK, DIM), jnp.float32),
            pltpu.VMEM((DIM,), jnp.float32),
            pltpu.VMEM((tw,), jnp.int32),
            pltpu.SMEM((CHUNK,), jnp.float32),
            pltpu.VMEM((S * DIM,), jnp.float32),
            pltpu.VMEM((16,), jnp.float32),
            pltpu.VMEM((16,), jnp.float32),
            pltpu.SemaphoreType.DMA,
            pltpu.SemaphoreType.DMA,
        ],
        compiler_params=pltpu.CompilerParams(needs_layout_passes=False),
    )


def _tc_body(x_ref, att_ref, idr_ref, m_out, d_out, acc_out,
             m_ref, d_ref, acc_ref):
    i = pl.program_id(0)
    nb = pl.num_programs(0)
    T = BLOCK_T

    @pl.when(i == 0)
    def _init():
        m_ref[...] = jnp.full((S, 1), NEG, jnp.float32)
        d_ref[...] = jnp.zeros((S, 1), jnp.float32)
        acc_ref[...] = jnp.zeros((S, DIM), jnp.float32)

    x = x_ref[...]
    a = att_ref[...]
    idr = idr_ref[0]

    l = lax.dot_general(a, x, (((1,), (1,)), ((), ())),
                        preferred_element_type=jnp.float32)
    seg_st = lax.broadcasted_iota(jnp.int32, (S, T), 0)
    mask = seg_st == idr
    lm = jnp.where(mask, l, NEG)
    bm = jnp.max(lm, axis=1, keepdims=True)
    m_old = m_ref[...]
    m_new = jnp.maximum(m_old, bm)
    c = jnp.exp(m_old - m_new)
    pw = jnp.exp(jnp.where(mask, l - m_new, NEG))
    d_ref[...] = d_ref[...] * c + jnp.sum(pw, axis=1, keepdims=True)
    m_ref[...] = m_new
    acc_ref[...] = (acc_ref[...] * c
                    + jnp.dot(pw, x, preferred_element_type=jnp.float32))

    @pl.when(i == nb - 1)
    def _fin():
        m_out[...] = m_ref[...]
        d_out[...] = d_ref[...]
        acc_out[...] = acc_ref[...]


def _tc_partial(seq_tc, att_row, idr):
    nb = seq_tc.shape[0] // BLOCK_T
    return pl.pallas_call(
        _tc_body,
        grid=(nb,),
        in_specs=[
            pl.BlockSpec((BLOCK_T, DIM), lambda i: (i, 0)),
            pl.BlockSpec((1, DIM), lambda i: (0, 0)),
            pl.BlockSpec((1, 1, BLOCK_T), lambda i: (i, 0, 0)),
        ],
        out_specs=[
            pl.BlockSpec((S, 1), lambda i: (0, 0)),
            pl.BlockSpec((S, 1), lambda i: (0, 0)),
            pl.BlockSpec((S, DIM), lambda i: (0, 0)),
        ],
        out_shape=[
            jax.ShapeDtypeStruct((S, 1), jnp.float32),
            jax.ShapeDtypeStruct((S, 1), jnp.float32),
            jax.ShapeDtypeStruct((S, DIM), jnp.float32),
        ],
        scratch_shapes=[
            pltpu.VMEM((S, 1), jnp.float32),
            pltpu.VMEM((S, 1), jnp.float32),
            pltpu.VMEM((S, DIM), jnp.float32),
        ],
        compiler_params=pltpu.CompilerParams(
            dimension_semantics=("arbitrary",)),
    )(seq_tc, att_row, idr)


def _merge_body(msc_ref, dsc_ref, accsc_ref, mtc_ref, dtc_ref, acctc_ref,
                out_ref):
    eye = (lax.broadcasted_iota(jnp.int32, (S, S), 0)
           == lax.broadcasted_iota(jnp.int32, (S, S), 1))
    ones_row = jnp.ones((1, S), jnp.float32)
    mtc_row = jnp.dot(ones_row, jnp.where(eye, mtc_ref[...], 0.0),
                      preferred_element_type=jnp.float32)
    dtc_row = jnp.dot(ones_row, jnp.where(eye, dtc_ref[...], 0.0),
                      preferred_element_type=jnp.float32)
    m_sc = msc_ref[...]
    M = jnp.maximum(jnp.max(m_sc, axis=0, keepdims=True), mtc_row)
    sc_sc = jnp.exp(m_sc - M)
    sc_tc = jnp.exp(mtc_row - M)
    D = (jnp.sum(dsc_ref[...] * sc_sc, axis=0, keepdims=True)
         + dtc_row * sc_tc)

    acc0 = jnp.dot(jnp.where(eye, sc_tc, 0.0), acctc_ref[...],
                   preferred_element_type=jnp.float32)

    def w_body(w, acc):
        mw = msc_ref[pl.ds(w, 1), :]
        scw = jnp.exp(mw - M)
        return acc + jnp.dot(jnp.where(eye, scw, 0.0),
                             accsc_ref[pl.ds(w * 16, 16), :],
                             preferred_element_type=jnp.float32)

    acc = lax.fori_loop(0, NW, w_body, acc0)
    dinv = jnp.where(eye, jnp.where(D > 0, 1.0 / D, 0.0), 0.0)
    out_ref[...] = jnp.dot(dinv, acc, preferred_element_type=jnp.float32)


def _merge(m_sc, d_sc, acc_sc, m_tc, d_tc, acc_tc):
    return pl.pallas_call(
        _merge_body,
        out_shape=jax.ShapeDtypeStruct((S, DIM), jnp.float32),
    )(m_sc, d_sc, acc_sc.reshape(NW * S, DIM), m_tc, d_tc, acc_tc)


@jax.jit
def kernel(seq, att, segment_ids):
    ids = segment_ids.astype(jnp.int32)
    att_row = att.reshape(1, DIM)
    att_flat = att.reshape(DIM)

    if K_TC > 0:
        idr = ids[:K_TC].reshape(K_TC // BLOCK_T, 1, BLOCK_T)
        m_tc, d_tc, acc_tc = _tc_partial(seq[:K_TC], att_row, idr)
    else:
        m_tc = jnp.full((S, 1), NEG, jnp.float32)
        d_tc = jnp.zeros((S, 1), jnp.float32)
        acc_tc = jnp.zeros((S, DIM), jnp.float32)

    tokens_sc = TOTAL_TOKENS - K_TC
    if tokens_sc > 0:
        m_sc, d_sc, acc_sc = _make_sc_call(tokens_sc)(
            seq[K_TC:], att_flat, ids[K_TC:])
    else:
        m_sc = jnp.full((NW, 16), NEG, jnp.float32)
        d_sc = jnp.zeros((NW, 16), jnp.float32)
        acc_sc = jnp.zeros((NW, S * DIM), jnp.float32)

    return _merge(m_sc, d_sc, acc_sc, m_tc, d_tc, acc_tc)


# speedup vs baseline: 11.8835x; 3.3733x over previous
import functools

import jax
import jax.numpy as jnp
from jax.experimental import pallas as pl
from jax.experimental.pallas import tpu as pltpu

NUM_SEGMENTS = 16
TOTAL_TOKENS = 32768
DIM = 1024
BLOCK_T = 4096
NEG = -1e30


def _body(x_ref, att_ref, idr_ref, out_ref, m_ref, d_ref, acc_ref):
    i = pl.program_id(0)
    nb = pl.num_programs(0)
    S = NUM_SEGMENTS
    T = BLOCK_T

    @pl.when(i == 0)
    def _init():
        m_ref[...] = jnp.full((S, 1), NEG, jnp.float32)
        d_ref[...] = jnp.zeros((S, 1), jnp.float32)
        acc_ref[...] = jnp.zeros((S, DIM), jnp.float32)

    x = x_ref[...]
    a = att_ref[...]
    idr = idr_ref[0]

    l = jax.lax.dot_general(a, x, (((1,), (1,)), ((), ())),
                            preferred_element_type=jnp.float32)

    seg_st = jax.lax.broadcasted_iota(jnp.int32, (S, T), 0)
    mask = seg_st == idr
    lm = jnp.where(mask, l, NEG)
    bm = jnp.max(lm, axis=1, keepdims=True)
    m_old = m_ref[...]
    m_new = jnp.maximum(m_old, bm)
    c = jnp.exp(m_old - m_new)
    pw = jnp.exp(jnp.where(mask, l - m_new, NEG))
    d_ref[...] = d_ref[...] * c + jnp.sum(pw, axis=1, keepdims=True)
    m_ref[...] = m_new
    acc_ref[...] = (acc_ref[...] * c
                    + jnp.dot(pw, x, preferred_element_type=jnp.float32))

    @pl.when(i == nb - 1)
    def _fin():
        d = d_ref[...]
        out_ref[...] = jnp.where(d > 0, acc_ref[...] / d, 0.0)


@jax.jit
def kernel(seq, att, segment_ids):
    ids = segment_ids.astype(jnp.int32)
    nb = TOTAL_TOKENS // BLOCK_T
    idr = ids.reshape(nb, 1, BLOCK_T)
    att_row = att.reshape(1, DIM)
    return pl.pallas_call(
        _body,
        grid=(nb,),
        in_specs=[
            pl.BlockSpec((BLOCK_T, DIM), lambda i: (i, 0)),
            pl.BlockSpec((1, DIM), lambda i: (0, 0)),
            pl.BlockSpec((1, 1, BLOCK_T), lambda i: (i, 0, 0)),
        ],
        out_specs=pl.BlockSpec((NUM_SEGMENTS, DIM), lambda i: (0, 0)),
        out_shape=jax.ShapeDtypeStruct((NUM_SEGMENTS, DIM), jnp.float32),
        scratch_shapes=[
            pltpu.VMEM((NUM_SEGMENTS, 1), jnp.float32),
            pltpu.VMEM((NUM_SEGMENTS, 1), jnp.float32),
            pltpu.VMEM((NUM_SEGMENTS, DIM), jnp.float32),
        ],
        compiler_params=pltpu.CompilerParams(
            dimension_semantics=("arbitrary",)),
    )(seq, att_row, idr)
